# Initial kernel scaffold; baseline (speedup 1.0000x reference)
import jax
import jax.numpy as jnp
from jax.experimental import pallas as pl


def kernel(loc_pred, conf_pred, refined_anchors, ignore_flags_refined_anchor, targets):
    raise NotImplementedError("write your pallas kernel here")


# speedup vs baseline: 4.2303x; 4.2303x over previous
import jax
import jax.numpy as jnp
from jax.experimental import pallas as pl
from jax.experimental.pallas import tpu as pltpu

_NCLS = 81
_THR = 0.5
_NPR = 3
_V0, _V1 = 0.1, 0.2
_R = 128
_L = 128


def _loss_kernel(loc_ref, conf_ref, anc_ref, ign_ref, tgt_ref,
                 ll_ref, lc_ref, np_ref):
    A = _R * _L
    O = tgt_ref.shape[2]
    C = conf_ref.shape[2]

    pcx = anc_ref[0, 0]
    pcy = anc_ref[0, 1]
    pw = anc_ref[0, 2]
    ph = anc_ref[0, 3]
    px1 = pcx - pw * 0.5
    py1 = pcy - ph * 0.5
    px2 = pcx + pw * 0.5
    py2 = pcy + ph * 0.5
    area_p = (px2 - px1) * (py2 - py1)

    r_io = jax.lax.broadcasted_iota(jnp.int32, (_R, _L), 0)
    c_io = jax.lax.broadcasted_iota(jnp.int32, (_R, _L), 1)
    lin = c_io * _R + r_io

    tgt = tgt_ref[0]
    rx1 = tgt[0:1, :]
    ry1 = tgt[1:2, :]
    rx2 = tgt[2:3, :]
    ry2 = tgt[3:4, :]
    rlb = tgt[4:5, :]
    io_o = jax.lax.broadcasted_iota(jnp.int32, (1, O), 1)

    def ext(row, t):
        return jnp.sum(jnp.where(io_o == t, row, 0.0))

    sub8 = jax.lax.broadcasted_iota(jnp.int32, (8, 128), 0)
    ln128 = jax.lax.broadcasted_iota(jnp.int32, (8, 128), 1)

    def _match_step(t, car):
        bto, bti, bpiv = car
        x1 = ext(rx1, t)
        y1 = ext(ry1, t)
        x2 = ext(rx2, t)
        y2 = ext(ry2, t)
        iw = jnp.maximum(jnp.minimum(x2, px2) - jnp.maximum(x1, px1), 0.0)
        ih = jnp.maximum(jnp.minimum(y2, py2) - jnp.maximum(y1, py1), 0.0)
        inter = iw * ih
        area_t = (x2 - x1) * (y2 - y1)
        iou = inter / (area_t + area_p - inter)
        upd = iou > bto
        bti = jnp.where(upd, t, bti)
        bto = jnp.where(upd, iou, bto)
        bpo = jnp.max(iou)
        bpi = jnp.min(jnp.where(iou == bpo, lin, A))
        bpiv = jnp.where((sub8 == 0) & (ln128 == t), bpi, bpiv)
        return bto, bti, bpiv

    bto, bti, bpiv = jax.lax.fori_loop(
        0, O, _match_step,
        (jnp.full((_R, _L), -1.0, jnp.float32),
         jnp.zeros((_R, _L), jnp.int32),
         jnp.zeros((8, 128), jnp.int32)))

    def _force_step(t, car):
        bto, bti = car
        bpi_t = jnp.sum(jnp.where((sub8 == 0) & (ln128 == t), bpiv, 0))
        mask = lin == bpi_t
        return jnp.where(mask, 2.0, bto), jnp.where(mask, t, bti)

    bto, bti = jax.lax.fori_loop(0, O, _force_step, (bto, bti))

    def _gather_step(t, car):
        mx1, my1, mx2, my2, mlb = car
        sel = bti == t
        mx1 = jnp.where(sel, ext(rx1, t), mx1)
        my1 = jnp.where(sel, ext(ry1, t), my1)
        mx2 = jnp.where(sel, ext(rx2, t), mx2)
        my2 = jnp.where(sel, ext(ry2, t), my2)
        mlb = jnp.where(sel, ext(rlb, t), mlb)
        return mx1, my1, mx2, my2, mlb

    z = jnp.zeros((_R, _L), jnp.float32)
    mx1, my1, mx2, my2, mlb = jax.lax.fori_loop(
        0, O, _gather_step, (z, z, z, z, z))

    pos = bto >= _THR
    posf = pos.astype(jnp.float32)
    conf_t = jnp.where(pos, mlb.astype(jnp.int32), 0)

    g0 = ((mx1 + mx2) * 0.5 - pcx) / (_V0 * pw)
    g1 = ((my1 + my2) * 0.5 - pcy) / (_V0 * ph)
    g2 = jnp.log((mx2 - mx1) / pw) / _V1
    g3 = jnp.log((my2 - my1) / ph) / _V1

    def _sl1(d):
        ad = jnp.abs(d)
        return jnp.where(ad < 1.0, 0.5 * d * d, ad - 0.5)

    sl1 = (_sl1(loc_ref[0, 0] - g0) + _sl1(loc_ref[0, 1] - g1) +
           _sl1(loc_ref[0, 2] - g2) + _sl1(loc_ref[0, 3] - g3))
    ll_ref[0] = jnp.sum(sl1 * posf, keepdims=True)

    cc_io = jax.lax.broadcasted_iota(jnp.int32, (128, C), 1)

    def _ce_step(i, ce_mat):
        ch = conf_ref[0, pl.ds(i * 128, 128), :]
        rmax = jnp.max(ch, axis=1, keepdims=True)
        lse = jnp.log(jnp.sum(jnp.exp(ch - rmax), axis=1,
                              keepdims=True)) + rmax
        ct_col = jnp.sum(jnp.where(c_io == i, conf_t, 0),
                         axis=1, keepdims=True)
        gath = jnp.sum(jnp.where(cc_io == ct_col, ch, 0.0),
                       axis=1, keepdims=True)
        ce_col = lse - gath
        return jnp.where(c_io == i, ce_col, ce_mat)

    ce_mat = jax.lax.fori_loop(0, _R, _ce_step, z)

    ign = ign_ref[0]
    proxy = jnp.where(pos | (ign > 0), 0.0, ce_mat)

    num_pos = jnp.sum(pos.astype(jnp.int32))
    max_neg = jnp.sum((proxy > 0.0).astype(jnp.int32))
    num_neg = jnp.minimum(_NPR * num_pos, max_neg)

    bits = jax.lax.bitcast_convert_type(proxy, jnp.int32)

    def _val_step(i, cur):
        cand = cur | jnp.left_shift(jnp.int32(1), 30 - i)
        cnt = jnp.sum((bits >= cand).astype(jnp.int32))
        return jnp.where(cnt >= num_neg, cand, cur)

    tbits = jax.lax.fori_loop(0, 31, _val_step, jnp.int32(0))
    cnt_gt = jnp.sum((bits > tbits).astype(jnp.int32))
    need = num_neg - cnt_gt
    tie = bits == tbits

    def _idx_step(i, cur):
        cand = cur | jnp.left_shift(jnp.int32(1), 13 - i)
        cnt = jnp.sum((tie & (lin < cand)).astype(jnp.int32))
        return jnp.where(cnt < need, cand, cur)

    tidx = jax.lax.fori_loop(0, 14, _idx_step, jnp.int32(0))
    neg = (bits > tbits) | (tie & (lin <= tidx) & (need > 0))
    neg = neg & (num_neg > 0)

    lc_ref[0] = jnp.sum(jnp.where(pos | neg, ce_mat, 0.0), keepdims=True)
    np_ref[0] = jnp.sum(posf, keepdims=True)


def kernel(loc_pred, conf_pred, refined_anchors, ignore_flags_refined_anchor,
           targets):
    B, A, _ = loc_pred.shape
    O = targets.shape[1]
    locT = jnp.transpose(loc_pred, (0, 2, 1)).reshape(B, 4, _R, _L).swapaxes(2, 3)
    ancT = jnp.transpose(refined_anchors, (0, 2, 1)).reshape(B, 4, _R, _L).swapaxes(2, 3)
    ignR = ignore_flags_refined_anchor.reshape(B, _R, _L).swapaxes(1, 2)
    tgtT = jnp.transpose(targets, (0, 2, 1))

    out = pl.pallas_call(
        _loss_kernel,
        grid=(B,),
        in_specs=[
            pl.BlockSpec((1, 4, _R, _L), lambda b: (b, 0, 0, 0)),
            pl.BlockSpec((1, A, _NCLS), lambda b: (b, 0, 0)),
            pl.BlockSpec((1, 4, _R, _L), lambda b: (b, 0, 0, 0)),
            pl.BlockSpec((1, _R, _L), lambda b: (b, 0, 0)),
            pl.BlockSpec((1, 5, O), lambda b: (b, 0, 0)),
        ],
        out_specs=[
            pl.BlockSpec((1, 1, 1), lambda b: (b, 0, 0)),
            pl.BlockSpec((1, 1, 1), lambda b: (b, 0, 0)),
            pl.BlockSpec((1, 1, 1), lambda b: (b, 0, 0)),
        ],
        out_shape=[
            jax.ShapeDtypeStruct((B, 1, 1), jnp.float32),
            jax.ShapeDtypeStruct((B, 1, 1), jnp.float32),
            jax.ShapeDtypeStruct((B, 1, 1), jnp.float32),
        ],
        compiler_params=pltpu.CompilerParams(
            dimension_semantics=("parallel",),
        ),
    )(locT, conf_pred, ancT, ignR, tgtT)

    ll, lc, npos = out
    total = jnp.sum(npos)
    return (jnp.sum(ll) / total, jnp.sum(lc) / total)


# speedup vs baseline: 6.3136x; 1.4925x over previous
import jax
import jax.numpy as jnp
from jax.experimental import pallas as pl
from jax.experimental.pallas import tpu as pltpu

_NCLS = 81
_THR = 0.5
_NPR = 3
_V0, _V1 = 0.1, 0.2
_R = 128
_L = 128


def _loss_kernel(loc_ref, conf_ref, anc_ref, ign_ref, tgt_ref,
                 ll_ref, lc_ref, np_ref, bpis_ref):
    A = _R * _L
    O = tgt_ref.shape[2]
    C = conf_ref.shape[2]

    pcx = anc_ref[0, 0]
    pcy = anc_ref[0, 1]
    pw = anc_ref[0, 2]
    ph = anc_ref[0, 3]
    px1 = pcx - pw * 0.5
    py1 = pcy - ph * 0.5
    px2 = pcx + pw * 0.5
    py2 = pcy + ph * 0.5
    area_p = (px2 - px1) * (py2 - py1)

    r_io = jax.lax.broadcasted_iota(jnp.int32, (_R, _L), 0)
    c_io = jax.lax.broadcasted_iota(jnp.int32, (_R, _L), 1)
    lin = c_io * _R + r_io

    def _match_step(t, car):
        bto, bti = car
        x1 = tgt_ref[0, 0, t]
        y1 = tgt_ref[0, 1, t]
        x2 = tgt_ref[0, 2, t]
        y2 = tgt_ref[0, 3, t]
        iw = jnp.maximum(jnp.minimum(x2, px2) - jnp.maximum(x1, px1), 0.0)
        ih = jnp.maximum(jnp.minimum(y2, py2) - jnp.maximum(y1, py1), 0.0)
        inter = iw * ih
        area_t = (x2 - x1) * (y2 - y1)
        iou = inter / (area_t + area_p - inter)
        upd = iou > bto
        bti = jnp.where(upd, t, bti)
        bto = jnp.where(upd, iou, bto)
        bpo = jnp.max(iou)
        bpis_ref[t] = jnp.min(jnp.where(iou == bpo, lin, A))
        return bto, bti

    bto, bti = jax.lax.fori_loop(
        0, O, _match_step,
        (jnp.full((_R, _L), -1.0, jnp.float32),
         jnp.zeros((_R, _L), jnp.int32)))

    def _force_step(t, car):
        bto, bti = car
        mask = lin == bpis_ref[t]
        return jnp.where(mask, 2.0, bto), jnp.where(mask, t, bti)

    bto, bti = jax.lax.fori_loop(0, O, _force_step, (bto, bti))

    def _gather_step(t, car):
        mx1, my1, mx2, my2, mlb = car
        sel = bti == t
        mx1 = jnp.where(sel, tgt_ref[0, 0, t], mx1)
        my1 = jnp.where(sel, tgt_ref[0, 1, t], my1)
        mx2 = jnp.where(sel, tgt_ref[0, 2, t], mx2)
        my2 = jnp.where(sel, tgt_ref[0, 3, t], my2)
        mlb = jnp.where(sel, tgt_ref[0, 4, t], mlb)
        return mx1, my1, mx2, my2, mlb

    z = jnp.zeros((_R, _L), jnp.float32)
    mx1, my1, mx2, my2, mlb = jax.lax.fori_loop(
        0, O, _gather_step, (z, z, z, z, z))

    pos = bto >= _THR
    posf = pos.astype(jnp.float32)
    conf_t = jnp.where(pos, mlb.astype(jnp.int32), 0)

    g0 = ((mx1 + mx2) * 0.5 - pcx) / (_V0 * pw)
    g1 = ((my1 + my2) * 0.5 - pcy) / (_V0 * ph)
    g2 = jnp.log((mx2 - mx1) / pw) / _V1
    g3 = jnp.log((my2 - my1) / ph) / _V1

    def _sl1(d):
        ad = jnp.abs(d)
        return jnp.where(ad < 1.0, 0.5 * d * d, ad - 0.5)

    sl1 = (_sl1(loc_ref[0, 0] - g0) + _sl1(loc_ref[0, 1] - g1) +
           _sl1(loc_ref[0, 2] - g2) + _sl1(loc_ref[0, 3] - g3))
    ll_ref[0] = jnp.sum(sl1 * posf, keepdims=True)

    _CH = 512
    _SUB = _CH // _R
    cc_io = jax.lax.broadcasted_iota(jnp.int32, (_CH, C), 1)

    def _ce_step(i, ce_mat):
        ch = conf_ref[0, pl.ds(i * _CH, _CH), :]
        rmax = jnp.max(ch, axis=1, keepdims=True)
        lse = jnp.log(jnp.sum(jnp.exp(ch - rmax), axis=1,
                              keepdims=True)) + rmax
        cols = [jnp.sum(jnp.where(c_io == i * _SUB + j, conf_t, 0),
                        axis=1, keepdims=True) for j in range(_SUB)]
        ct_chunk = jnp.concatenate(cols, axis=0)
        gath = jnp.sum(jnp.where(cc_io == ct_chunk, ch, 0.0),
                       axis=1, keepdims=True)
        ce_col = lse - gath
        for j in range(_SUB):
            ce_mat = jnp.where(c_io == i * _SUB + j,
                               ce_col[j * _R:(j + 1) * _R, :], ce_mat)
        return ce_mat

    ce_mat = jax.lax.fori_loop(0, _R * _L // _CH, _ce_step, z)

    ign = ign_ref[0]
    proxy = jnp.where(pos | (ign > 0), 0.0, ce_mat)

    num_pos = jnp.sum(pos.astype(jnp.int32))
    max_neg = jnp.sum((proxy > 0.0).astype(jnp.int32))
    num_neg = jnp.minimum(_NPR * num_pos, max_neg)

    bits = jax.lax.bitcast_convert_type(proxy, jnp.int32)

    def _val_step(i, cur):
        cand = cur | jnp.left_shift(jnp.int32(1), 30 - i)
        cnt = jnp.sum((bits >= cand).astype(jnp.int32))
        return jnp.where(cnt >= num_neg, cand, cur)

    tbits = jax.lax.fori_loop(0, 31, _val_step, jnp.int32(0))
    cnt_gt = jnp.sum((bits > tbits).astype(jnp.int32))
    need = num_neg - cnt_gt
    tie = bits == tbits

    def _idx_step(i, cur):
        cand = cur | jnp.left_shift(jnp.int32(1), 13 - i)
        cnt = jnp.sum((tie & (lin < cand)).astype(jnp.int32))
        return jnp.where(cnt < need, cand, cur)

    tidx = jax.lax.fori_loop(0, 14, _idx_step, jnp.int32(0))
    neg = (bits > tbits) | (tie & (lin <= tidx) & (need > 0))
    neg = neg & (num_neg > 0)

    lc_ref[0] = jnp.sum(jnp.where(pos | neg, ce_mat, 0.0), keepdims=True)
    np_ref[0] = jnp.sum(posf, keepdims=True)


def kernel(loc_pred, conf_pred, refined_anchors, ignore_flags_refined_anchor,
           targets):
    B, A, _ = loc_pred.shape
    O = targets.shape[1]
    locT = jnp.transpose(loc_pred, (0, 2, 1)).reshape(B, 4, _R, _L).swapaxes(2, 3)
    ancT = jnp.transpose(refined_anchors, (0, 2, 1)).reshape(B, 4, _R, _L).swapaxes(2, 3)
    ignR = ignore_flags_refined_anchor.reshape(B, _R, _L).swapaxes(1, 2)
    tgtT = jnp.transpose(targets, (0, 2, 1))

    out = pl.pallas_call(
        _loss_kernel,
        grid=(B,),
        in_specs=[
            pl.BlockSpec((1, 4, _R, _L), lambda b: (b, 0, 0, 0)),
            pl.BlockSpec((1, A, _NCLS), lambda b: (b, 0, 0)),
            pl.BlockSpec((1, 4, _R, _L), lambda b: (b, 0, 0, 0)),
            pl.BlockSpec((1, _R, _L), lambda b: (b, 0, 0)),
            pl.BlockSpec((1, 5, O), lambda b: (b, 0, 0),
                         memory_space=pltpu.SMEM),
        ],
        out_specs=[
            pl.BlockSpec((1, 1, 1), lambda b: (b, 0, 0)),
            pl.BlockSpec((1, 1, 1), lambda b: (b, 0, 0)),
            pl.BlockSpec((1, 1, 1), lambda b: (b, 0, 0)),
        ],
        out_shape=[
            jax.ShapeDtypeStruct((B, 1, 1), jnp.float32),
            jax.ShapeDtypeStruct((B, 1, 1), jnp.float32),
            jax.ShapeDtypeStruct((B, 1, 1), jnp.float32),
        ],
        scratch_shapes=[pltpu.SMEM((O,), jnp.int32)],
        compiler_params=pltpu.CompilerParams(
            dimension_semantics=("parallel",),
        ),
    )(locT, conf_pred, ancT, ignR, tgtT)

    ll, lc, npos = out
    total = jnp.sum(npos)
    return (jnp.sum(ll) / total, jnp.sum(lc) / total)


# speedup vs baseline: 7.5996x; 1.2037x over previous
import jax
import jax.numpy as jnp
from jax.experimental import pallas as pl
from jax.experimental.pallas import tpu as pltpu

_NCLS = 81
_THR = 0.5
_NPR = 3
_V0, _V1 = 0.1, 0.2
_R = 128
_L = 128


def _loss_kernel(loc_ref, conf_ref, anc_ref, ign_ref, tgt_ref,
                 ll_ref, lc_ref, np_ref, bpis_ref):
    A = _R * _L
    O = tgt_ref.shape[2]
    C = conf_ref.shape[2]

    pcx = anc_ref[0, 0]
    pcy = anc_ref[0, 1]
    pw = anc_ref[0, 2]
    ph = anc_ref[0, 3]
    px1 = pcx - pw * 0.5
    py1 = pcy - ph * 0.5
    px2 = pcx + pw * 0.5
    py2 = pcy + ph * 0.5
    area_p = (px2 - px1) * (py2 - py1)

    r_io = jax.lax.broadcasted_iota(jnp.int32, (_R, _L), 0)
    c_io = jax.lax.broadcasted_iota(jnp.int32, (_R, _L), 1)
    lin = c_io * _R + r_io

    def _match_step(t, car):
        bto, bti = car
        x1 = tgt_ref[0, 0, t]
        y1 = tgt_ref[0, 1, t]
        x2 = tgt_ref[0, 2, t]
        y2 = tgt_ref[0, 3, t]
        iw = jnp.maximum(jnp.minimum(x2, px2) - jnp.maximum(x1, px1), 0.0)
        ih = jnp.maximum(jnp.minimum(y2, py2) - jnp.maximum(y1, py1), 0.0)
        inter = iw * ih
        area_t = (x2 - x1) * (y2 - y1)
        iou = inter / (area_t + area_p - inter)
        upd = iou > bto
        bti = jnp.where(upd, t, bti)
        bto = jnp.where(upd, iou, bto)
        bpo = jnp.max(iou)
        bpis_ref[t] = jnp.min(jnp.where(iou == bpo, lin, A))
        return bto, bti

    bto, bti = jax.lax.fori_loop(
        0, O, _match_step,
        (jnp.full((_R, _L), -1.0, jnp.float32),
         jnp.zeros((_R, _L), jnp.int32)), unroll=5)

    def _force_step(t, car):
        bto, bti = car
        mask = lin == bpis_ref[t]
        return jnp.where(mask, 2.0, bto), jnp.where(mask, t, bti)

    bto, bti = jax.lax.fori_loop(0, O, _force_step, (bto, bti), unroll=5)

    def _gather_step(t, car):
        mx1, my1, mx2, my2, mlb = car
        sel = bti == t
        mx1 = jnp.where(sel, tgt_ref[0, 0, t], mx1)
        my1 = jnp.where(sel, tgt_ref[0, 1, t], my1)
        mx2 = jnp.where(sel, tgt_ref[0, 2, t], mx2)
        my2 = jnp.where(sel, tgt_ref[0, 3, t], my2)
        mlb = jnp.where(sel, tgt_ref[0, 4, t], mlb)
        return mx1, my1, mx2, my2, mlb

    z = jnp.zeros((_R, _L), jnp.float32)
    mx1, my1, mx2, my2, mlb = jax.lax.fori_loop(
        0, O, _gather_step, (z, z, z, z, z), unroll=5)

    pos = bto >= _THR
    posf = pos.astype(jnp.float32)
    conf_t = jnp.where(pos, mlb.astype(jnp.int32), 0)

    g0 = ((mx1 + mx2) * 0.5 - pcx) / (_V0 * pw)
    g1 = ((my1 + my2) * 0.5 - pcy) / (_V0 * ph)
    g2 = jnp.log((mx2 - mx1) / pw) / _V1
    g3 = jnp.log((my2 - my1) / ph) / _V1

    def _sl1(d):
        ad = jnp.abs(d)
        return jnp.where(ad < 1.0, 0.5 * d * d, ad - 0.5)

    sl1 = (_sl1(loc_ref[0, 0] - g0) + _sl1(loc_ref[0, 1] - g1) +
           _sl1(loc_ref[0, 2] - g2) + _sl1(loc_ref[0, 3] - g3))
    ll_ref[0] = jnp.sum(sl1 * posf, keepdims=True)

    _CH = 512
    _SUB = _CH // _R
    cc_io = jax.lax.broadcasted_iota(jnp.int32, (_CH, C), 1)

    def _ce_step(i, ce_mat):
        ch = conf_ref[0, pl.ds(i * _CH, _CH), :]
        rmax = jnp.max(ch, axis=1, keepdims=True)
        lse = jnp.log(jnp.sum(jnp.exp(ch - rmax), axis=1,
                              keepdims=True)) + rmax
        cols = [jnp.sum(jnp.where(c_io == i * _SUB + j, conf_t, 0),
                        axis=1, keepdims=True) for j in range(_SUB)]
        ct_chunk = jnp.concatenate(cols, axis=0)
        gath = jnp.sum(jnp.where(cc_io == ct_chunk, ch, 0.0),
                       axis=1, keepdims=True)
        ce_col = lse - gath
        for j in range(_SUB):
            ce_mat = jnp.where(c_io == i * _SUB + j,
                               ce_col[j * _R:(j + 1) * _R, :], ce_mat)
        return ce_mat

    ce_mat = jax.lax.fori_loop(0, _R * _L // _CH, _ce_step, z, unroll=2)

    ign = ign_ref[0]
    proxy = jnp.where(pos | (ign > 0), 0.0, ce_mat)

    num_pos = jnp.sum(pos.astype(jnp.int32))
    max_neg = jnp.sum((proxy > 0.0).astype(jnp.int32))
    num_neg = jnp.minimum(_NPR * num_pos, max_neg)

    bits = jax.lax.bitcast_convert_type(proxy, jnp.int32)

    def _val_step(i, cur):
        cand = cur | jnp.left_shift(jnp.int32(1), 30 - i)
        cnt = jnp.sum((bits >= cand).astype(jnp.int32))
        return jnp.where(cnt >= num_neg, cand, cur)

    tbits = jax.lax.fori_loop(0, 31, _val_step, jnp.int32(0))
    cnt_gt = jnp.sum((bits > tbits).astype(jnp.int32))
    need = num_neg - cnt_gt
    tie = bits == tbits

    def _idx_step(i, cur):
        cand = cur | jnp.left_shift(jnp.int32(1), 13 - i)
        cnt = jnp.sum((tie & (lin < cand)).astype(jnp.int32))
        return jnp.where(cnt < need, cand, cur)

    tidx = jax.lax.fori_loop(0, 14, _idx_step, jnp.int32(0))
    neg = (bits > tbits) | (tie & (lin <= tidx) & (need > 0))
    neg = neg & (num_neg > 0)

    lc_ref[0] = jnp.sum(jnp.where(pos | neg, ce_mat, 0.0), keepdims=True)
    np_ref[0] = jnp.sum(posf, keepdims=True)


def kernel(loc_pred, conf_pred, refined_anchors, ignore_flags_refined_anchor,
           targets):
    B, A, _ = loc_pred.shape
    O = targets.shape[1]
    locT = jnp.transpose(loc_pred, (0, 2, 1)).reshape(B, 4, _R, _L).swapaxes(2, 3)
    ancT = jnp.transpose(refined_anchors, (0, 2, 1)).reshape(B, 4, _R, _L).swapaxes(2, 3)
    ignR = ignore_flags_refined_anchor.reshape(B, _R, _L).swapaxes(1, 2)
    tgtT = jnp.transpose(targets, (0, 2, 1))

    out = pl.pallas_call(
        _loss_kernel,
        grid=(B,),
        in_specs=[
            pl.BlockSpec((1, 4, _R, _L), lambda b: (b, 0, 0, 0)),
            pl.BlockSpec((1, A, _NCLS), lambda b: (b, 0, 0)),
            pl.BlockSpec((1, 4, _R, _L), lambda b: (b, 0, 0, 0)),
            pl.BlockSpec((1, _R, _L), lambda b: (b, 0, 0)),
            pl.BlockSpec((1, 5, O), lambda b: (b, 0, 0),
                         memory_space=pltpu.SMEM),
        ],
        out_specs=[
            pl.BlockSpec((1, 1, 1), lambda b: (b, 0, 0)),
            pl.BlockSpec((1, 1, 1), lambda b: (b, 0, 0)),
            pl.BlockSpec((1, 1, 1), lambda b: (b, 0, 0)),
        ],
        out_shape=[
            jax.ShapeDtypeStruct((B, 1, 1), jnp.float32),
            jax.ShapeDtypeStruct((B, 1, 1), jnp.float32),
            jax.ShapeDtypeStruct((B, 1, 1), jnp.float32),
        ],
        scratch_shapes=[pltpu.SMEM((O,), jnp.int32)],
        compiler_params=pltpu.CompilerParams(
            dimension_semantics=("parallel",),
        ),
    )(locT, conf_pred, ancT, ignR, tgtT)

    ll, lc, npos = out
    total = jnp.sum(npos)
    return (jnp.sum(ll) / total, jnp.sum(lc) / total)


# speedup vs baseline: 7.9641x; 1.0480x over previous
import jax
import jax.numpy as jnp
from jax.experimental import pallas as pl
from jax.experimental.pallas import tpu as pltpu

_NCLS = 81
_THR = 0.5
_NPR = 3
_V0, _V1 = 0.1, 0.2
_R = 128
_L = 128


def _loss_kernel(loc_ref, conf_ref, anc_ref, ign_ref, tgt_ref,
                 ll_ref, lc_ref, np_ref, bpis_ref):
    A = _R * _L
    O = tgt_ref.shape[2]
    C = conf_ref.shape[2]

    pcx = anc_ref[0, 0]
    pcy = anc_ref[0, 1]
    pw = anc_ref[0, 2]
    ph = anc_ref[0, 3]
    px1 = pcx - pw * 0.5
    py1 = pcy - ph * 0.5
    px2 = pcx + pw * 0.5
    py2 = pcy + ph * 0.5
    area_p = (px2 - px1) * (py2 - py1)

    r_io = jax.lax.broadcasted_iota(jnp.int32, (_R, _L), 0)
    c_io = jax.lax.broadcasted_iota(jnp.int32, (_R, _L), 1)
    lin = c_io * _R + r_io

    def _match_step(t, car):
        bto, bti = car
        x1 = tgt_ref[0, 0, t]
        y1 = tgt_ref[0, 1, t]
        x2 = tgt_ref[0, 2, t]
        y2 = tgt_ref[0, 3, t]
        iw = jnp.maximum(jnp.minimum(x2, px2) - jnp.maximum(x1, px1), 0.0)
        ih = jnp.maximum(jnp.minimum(y2, py2) - jnp.maximum(y1, py1), 0.0)
        inter = iw * ih
        area_t = (x2 - x1) * (y2 - y1)
        iou = inter / (area_t + area_p - inter)
        upd = iou > bto
        bti = jnp.where(upd, t, bti)
        bto = jnp.where(upd, iou, bto)
        bpo = jnp.max(iou)
        bpis_ref[t] = jnp.min(jnp.where(iou == bpo, lin, A))
        return bto, bti

    bto, bti = jax.lax.fori_loop(
        0, O, _match_step,
        (jnp.full((_R, _L), -1.0, jnp.float32),
         jnp.zeros((_R, _L), jnp.int32)), unroll=10)

    def _force_step(t, car):
        bto, bti = car
        mask = lin == bpis_ref[t]
        return jnp.where(mask, 2.0, bto), jnp.where(mask, t, bti)

    bto, bti = jax.lax.fori_loop(0, O, _force_step, (bto, bti), unroll=10)

    def _gather_step(t, car):
        mx1, my1, mx2, my2, mlb = car
        sel = bti == t
        mx1 = jnp.where(sel, tgt_ref[0, 0, t], mx1)
        my1 = jnp.where(sel, tgt_ref[0, 1, t], my1)
        mx2 = jnp.where(sel, tgt_ref[0, 2, t], mx2)
        my2 = jnp.where(sel, tgt_ref[0, 3, t], my2)
        mlb = jnp.where(sel, tgt_ref[0, 4, t], mlb)
        return mx1, my1, mx2, my2, mlb

    z = jnp.zeros((_R, _L), jnp.float32)
    mx1, my1, mx2, my2, mlb = jax.lax.fori_loop(
        0, O, _gather_step, (z, z, z, z, z), unroll=10)

    pos = bto >= _THR
    posf = pos.astype(jnp.float32)
    conf_t = jnp.where(pos, mlb.astype(jnp.int32), 0)

    g0 = ((mx1 + mx2) * 0.5 - pcx) / (_V0 * pw)
    g1 = ((my1 + my2) * 0.5 - pcy) / (_V0 * ph)
    g2 = jnp.log((mx2 - mx1) / pw) / _V1
    g3 = jnp.log((my2 - my1) / ph) / _V1

    def _sl1(d):
        ad = jnp.abs(d)
        return jnp.where(ad < 1.0, 0.5 * d * d, ad - 0.5)

    sl1 = (_sl1(loc_ref[0, 0] - g0) + _sl1(loc_ref[0, 1] - g1) +
           _sl1(loc_ref[0, 2] - g2) + _sl1(loc_ref[0, 3] - g3))
    ll_ref[0] = jnp.sum(sl1 * posf, keepdims=True)

    _CH = 512
    _SUB = _CH // _R
    cc_io = jax.lax.broadcasted_iota(jnp.int32, (_CH, C), 1)

    def _ce_step(i, ce_mat):
        ch = conf_ref[0, pl.ds(i * _CH, _CH), :]
        rmax = jnp.max(ch, axis=1, keepdims=True)
        lse = jnp.log(jnp.sum(jnp.exp(ch - rmax), axis=1,
                              keepdims=True)) + rmax
        cols = [jnp.sum(jnp.where(c_io == i * _SUB + j, conf_t, 0),
                        axis=1, keepdims=True) for j in range(_SUB)]
        ct_chunk = jnp.concatenate(cols, axis=0)
        gath = jnp.sum(jnp.where(cc_io == ct_chunk, ch, 0.0),
                       axis=1, keepdims=True)
        ce_col = lse - gath
        for j in range(_SUB):
            ce_mat = jnp.where(c_io == i * _SUB + j,
                               ce_col[j * _R:(j + 1) * _R, :], ce_mat)
        return ce_mat

    ce_mat = jax.lax.fori_loop(0, _R * _L // _CH, _ce_step, z, unroll=4)

    ign = ign_ref[0]
    proxy = jnp.where(pos | (ign > 0), 0.0, ce_mat)

    num_pos = jnp.sum(pos.astype(jnp.int32))
    max_neg = jnp.sum((proxy > 0.0).astype(jnp.int32))
    num_neg = jnp.minimum(_NPR * num_pos, max_neg)

    bits = jax.lax.bitcast_convert_type(proxy, jnp.int32)

    def _val_step(i, cur):
        cand = cur | jnp.left_shift(jnp.int32(1), 30 - i)
        cnt = jnp.sum((bits >= cand).astype(jnp.int32))
        return jnp.where(cnt >= num_neg, cand, cur)

    tbits = jax.lax.fori_loop(0, 31, _val_step, jnp.int32(0), unroll=31)
    cnt_gt = jnp.sum((bits > tbits).astype(jnp.int32))
    need = num_neg - cnt_gt
    tie = bits == tbits

    def _idx_step(i, cur):
        cand = cur | jnp.left_shift(jnp.int32(1), 13 - i)
        cnt = jnp.sum((tie & (lin < cand)).astype(jnp.int32))
        return jnp.where(cnt < need, cand, cur)

    tidx = jax.lax.fori_loop(0, 14, _idx_step, jnp.int32(0), unroll=14)
    neg = (bits > tbits) | (tie & (lin <= tidx) & (need > 0))
    neg = neg & (num_neg > 0)

    lc_ref[0] = jnp.sum(jnp.where(pos | neg, ce_mat, 0.0), keepdims=True)
    np_ref[0] = jnp.sum(posf, keepdims=True)


def kernel(loc_pred, conf_pred, refined_anchors, ignore_flags_refined_anchor,
           targets):
    B, A, _ = loc_pred.shape
    O = targets.shape[1]
    locT = jnp.transpose(loc_pred, (0, 2, 1)).reshape(B, 4, _R, _L).swapaxes(2, 3)
    ancT = jnp.transpose(refined_anchors, (0, 2, 1)).reshape(B, 4, _R, _L).swapaxes(2, 3)
    ignR = ignore_flags_refined_anchor.reshape(B, _R, _L).swapaxes(1, 2)
    tgtT = jnp.transpose(targets, (0, 2, 1))

    out = pl.pallas_call(
        _loss_kernel,
        grid=(B,),
        in_specs=[
            pl.BlockSpec((1, 4, _R, _L), lambda b: (b, 0, 0, 0)),
            pl.BlockSpec((1, A, _NCLS), lambda b: (b, 0, 0)),
            pl.BlockSpec((1, 4, _R, _L), lambda b: (b, 0, 0, 0)),
            pl.BlockSpec((1, _R, _L), lambda b: (b, 0, 0)),
            pl.BlockSpec((1, 5, O), lambda b: (b, 0, 0),
                         memory_space=pltpu.SMEM),
        ],
        out_specs=[
            pl.BlockSpec((1, 1, 1), lambda b: (b, 0, 0)),
            pl.BlockSpec((1, 1, 1), lambda b: (b, 0, 0)),
            pl.BlockSpec((1, 1, 1), lambda b: (b, 0, 0)),
        ],
        out_shape=[
            jax.ShapeDtypeStruct((B, 1, 1), jnp.float32),
            jax.ShapeDtypeStruct((B, 1, 1), jnp.float32),
            jax.ShapeDtypeStruct((B, 1, 1), jnp.float32),
        ],
        scratch_shapes=[pltpu.SMEM((O,), jnp.int32)],
        compiler_params=pltpu.CompilerParams(
            dimension_semantics=("parallel",),
        ),
    )(locT, conf_pred, ancT, ignR, tgtT)

    ll, lc, npos = out
    total = jnp.sum(npos)
    return (jnp.sum(ll) / total, jnp.sum(lc) / total)


# speedup vs baseline: 8.1951x; 1.0290x over previous
import jax
import jax.numpy as jnp
from jax.experimental import pallas as pl
from jax.experimental.pallas import tpu as pltpu

_NCLS = 81
_THR = 0.5
_NPR = 3
_V0, _V1 = 0.1, 0.2
_R = 128
_L = 128


def _loss_kernel(loc_ref, conf_ref, anc_ref, ign_ref, tgt_ref,
                 ll_ref, lc_ref, np_ref, bpis_ref):
    A = _R * _L
    O = tgt_ref.shape[2]
    C = conf_ref.shape[2]

    pcx = anc_ref[0, 0]
    pcy = anc_ref[0, 1]
    pw = anc_ref[0, 2]
    ph = anc_ref[0, 3]
    px1 = pcx - pw * 0.5
    py1 = pcy - ph * 0.5
    px2 = pcx + pw * 0.5
    py2 = pcy + ph * 0.5
    area_p = (px2 - px1) * (py2 - py1)

    r_io = jax.lax.broadcasted_iota(jnp.int32, (_R, _L), 0)
    c_io = jax.lax.broadcasted_iota(jnp.int32, (_R, _L), 1)
    lin = c_io * _R + r_io

    def _match_step(t, car):
        bto, bti = car
        x1 = tgt_ref[0, 0, t]
        y1 = tgt_ref[0, 1, t]
        x2 = tgt_ref[0, 2, t]
        y2 = tgt_ref[0, 3, t]
        iw = jnp.maximum(jnp.minimum(x2, px2) - jnp.maximum(x1, px1), 0.0)
        ih = jnp.maximum(jnp.minimum(y2, py2) - jnp.maximum(y1, py1), 0.0)
        inter = iw * ih
        area_t = (x2 - x1) * (y2 - y1)
        iou = inter / (area_t + area_p - inter)
        upd = iou > bto
        bti = jnp.where(upd, t, bti)
        bto = jnp.where(upd, iou, bto)
        bpo = jnp.max(iou)
        bpis_ref[t] = jnp.min(jnp.where(iou == bpo, lin, A))
        return bto, bti

    bto, bti = jax.lax.fori_loop(
        0, O, _match_step,
        (jnp.full((_R, _L), -1.0, jnp.float32),
         jnp.zeros((_R, _L), jnp.int32)), unroll=10)

    def _force_step(t, car):
        bto, bti = car
        mask = lin == bpis_ref[t]
        return jnp.where(mask, 2.0, bto), jnp.where(mask, t, bti)

    bto, bti = jax.lax.fori_loop(0, O, _force_step, (bto, bti), unroll=10)

    def _gather_step(t, car):
        mx1, my1, mx2, my2, mlb = car
        sel = bti == t
        mx1 = jnp.where(sel, tgt_ref[0, 0, t], mx1)
        my1 = jnp.where(sel, tgt_ref[0, 1, t], my1)
        mx2 = jnp.where(sel, tgt_ref[0, 2, t], mx2)
        my2 = jnp.where(sel, tgt_ref[0, 3, t], my2)
        mlb = jnp.where(sel, tgt_ref[0, 4, t], mlb)
        return mx1, my1, mx2, my2, mlb

    z = jnp.zeros((_R, _L), jnp.float32)
    mx1, my1, mx2, my2, mlb = jax.lax.fori_loop(
        0, O, _gather_step, (z, z, z, z, z), unroll=10)

    pos = bto >= _THR
    posf = pos.astype(jnp.float32)
    conf_t = jnp.where(pos, mlb.astype(jnp.int32), 0)

    g0 = ((mx1 + mx2) * 0.5 - pcx) / (_V0 * pw)
    g1 = ((my1 + my2) * 0.5 - pcy) / (_V0 * ph)
    g2 = jnp.log((mx2 - mx1) / pw) / _V1
    g3 = jnp.log((my2 - my1) / ph) / _V1

    def _sl1(d):
        ad = jnp.abs(d)
        return jnp.where(ad < 1.0, 0.5 * d * d, ad - 0.5)

    sl1 = (_sl1(loc_ref[0, 0] - g0) + _sl1(loc_ref[0, 1] - g1) +
           _sl1(loc_ref[0, 2] - g2) + _sl1(loc_ref[0, 3] - g3))
    ll_ref[0] = jnp.sum(sl1 * posf, keepdims=True)

    _CH = 1024
    _SUB = _CH // _R
    cc_io = jax.lax.broadcasted_iota(jnp.int32, (_CH, C), 1)

    def _ce_step(i, ce_mat):
        ch = conf_ref[0, pl.ds(i * _CH, _CH), :]
        rmax = jnp.max(ch, axis=1, keepdims=True)
        lse = jnp.log(jnp.sum(jnp.exp(ch - rmax), axis=1,
                              keepdims=True)) + rmax
        cols = [jnp.sum(jnp.where(c_io == i * _SUB + j, conf_t, 0),
                        axis=1, keepdims=True) for j in range(_SUB)]
        ct_chunk = jnp.concatenate(cols, axis=0)
        gath = jnp.sum(jnp.where(cc_io == ct_chunk, ch, 0.0),
                       axis=1, keepdims=True)
        ce_col = lse - gath
        for j in range(_SUB):
            ce_mat = jnp.where(c_io == i * _SUB + j,
                               ce_col[j * _R:(j + 1) * _R, :], ce_mat)
        return ce_mat

    ce_mat = jax.lax.fori_loop(0, _R * _L // _CH, _ce_step, z, unroll=2)

    ign = ign_ref[0]
    proxy = jnp.where(pos | (ign > 0), 0.0, ce_mat)

    num_pos = jnp.sum(pos.astype(jnp.int32))
    max_neg = jnp.sum((proxy > 0.0).astype(jnp.int32))
    num_neg = jnp.minimum(_NPR * num_pos, max_neg)

    bits = jax.lax.bitcast_convert_type(proxy, jnp.int32)

    def _val_step(i, cur):
        cand = cur | jnp.left_shift(jnp.int32(1), 30 - i)
        cnt = jnp.sum((bits >= cand).astype(jnp.int32))
        return jnp.where(cnt >= num_neg, cand, cur)

    tbits = jax.lax.fori_loop(0, 31, _val_step, jnp.int32(0), unroll=31)
    cnt_gt = jnp.sum((bits > tbits).astype(jnp.int32))
    need = num_neg - cnt_gt
    tie = bits == tbits

    def _idx_step(i, cur):
        cand = cur | jnp.left_shift(jnp.int32(1), 13 - i)
        cnt = jnp.sum((tie & (lin < cand)).astype(jnp.int32))
        return jnp.where(cnt < need, cand, cur)

    tidx = jax.lax.fori_loop(0, 14, _idx_step, jnp.int32(0), unroll=14)
    neg = (bits > tbits) | (tie & (lin <= tidx) & (need > 0))
    neg = neg & (num_neg > 0)

    lc_ref[0] = jnp.sum(jnp.where(pos | neg, ce_mat, 0.0), keepdims=True)
    np_ref[0] = jnp.sum(posf, keepdims=True)


def kernel(loc_pred, conf_pred, refined_anchors, ignore_flags_refined_anchor,
           targets):
    B, A, _ = loc_pred.shape
    O = targets.shape[1]
    locT = jnp.transpose(loc_pred, (0, 2, 1)).reshape(B, 4, _R, _L).swapaxes(2, 3)
    ancT = jnp.transpose(refined_anchors, (0, 2, 1)).reshape(B, 4, _R, _L).swapaxes(2, 3)
    ignR = ignore_flags_refined_anchor.reshape(B, _R, _L).swapaxes(1, 2)
    tgtT = jnp.transpose(targets, (0, 2, 1))

    out = pl.pallas_call(
        _loss_kernel,
        grid=(B,),
        in_specs=[
            pl.BlockSpec((1, 4, _R, _L), lambda b: (b, 0, 0, 0)),
            pl.BlockSpec((1, A, _NCLS), lambda b: (b, 0, 0)),
            pl.BlockSpec((1, 4, _R, _L), lambda b: (b, 0, 0, 0)),
            pl.BlockSpec((1, _R, _L), lambda b: (b, 0, 0)),
            pl.BlockSpec((1, 5, O), lambda b: (b, 0, 0),
                         memory_space=pltpu.SMEM),
        ],
        out_specs=[
            pl.BlockSpec((1, 1, 1), lambda b: (b, 0, 0)),
            pl.BlockSpec((1, 1, 1), lambda b: (b, 0, 0)),
            pl.BlockSpec((1, 1, 1), lambda b: (b, 0, 0)),
        ],
        out_shape=[
            jax.ShapeDtypeStruct((B, 1, 1), jnp.float32),
            jax.ShapeDtypeStruct((B, 1, 1), jnp.float32),
            jax.ShapeDtypeStruct((B, 1, 1), jnp.float32),
        ],
        scratch_shapes=[pltpu.SMEM((O,), jnp.int32)],
        compiler_params=pltpu.CompilerParams(
            dimension_semantics=("parallel",),
        ),
    )(locT, conf_pred, ancT, ignR, tgtT)

    ll, lc, npos = out
    total = jnp.sum(npos)
    return (jnp.sum(ll) / total, jnp.sum(lc) / total)


# speedup vs baseline: 8.2944x; 1.0121x over previous
import jax
import jax.numpy as jnp
from jax.experimental import pallas as pl
from jax.experimental.pallas import tpu as pltpu

_NCLS = 81
_THR = 0.5
_NPR = 3
_V0, _V1 = 0.1, 0.2
_R = 128
_L = 128


def _loss_kernel(loc_ref, conf_ref, anc_ref, ign_ref, tgt_ref,
                 ll_ref, lc_ref, np_ref, bpis_ref):
    A = _R * _L
    O = tgt_ref.shape[2]
    C = conf_ref.shape[2]

    pcx = anc_ref[0, 0]
    pcy = anc_ref[0, 1]
    pw = anc_ref[0, 2]
    ph = anc_ref[0, 3]
    px1 = pcx - pw * 0.5
    py1 = pcy - ph * 0.5
    px2 = pcx + pw * 0.5
    py2 = pcy + ph * 0.5
    area_p = (px2 - px1) * (py2 - py1)

    r_io = jax.lax.broadcasted_iota(jnp.int32, (_R, _L), 0)
    c_io = jax.lax.broadcasted_iota(jnp.int32, (_R, _L), 1)
    lin = c_io * _R + r_io

    def _match_step(t, car):
        bto, bti = car
        x1 = tgt_ref[0, 0, t]
        y1 = tgt_ref[0, 1, t]
        x2 = tgt_ref[0, 2, t]
        y2 = tgt_ref[0, 3, t]
        iw = jnp.maximum(jnp.minimum(x2, px2) - jnp.maximum(x1, px1), 0.0)
        ih = jnp.maximum(jnp.minimum(y2, py2) - jnp.maximum(y1, py1), 0.0)
        inter = iw * ih
        area_t = (x2 - x1) * (y2 - y1)
        iou = inter / (area_t + area_p - inter)
        upd = iou > bto
        bti = jnp.where(upd, t, bti)
        bto = jnp.where(upd, iou, bto)
        bpo = jnp.max(iou)
        bpis_ref[t] = jnp.min(jnp.where(iou == bpo, lin, A))
        return bto, bti

    bto, bti = jax.lax.fori_loop(
        0, O, _match_step,
        (jnp.full((_R, _L), -1.0, jnp.float32),
         jnp.zeros((_R, _L), jnp.int32)), unroll=25)

    def _force_step(t, car):
        bto, bti = car
        mask = lin == bpis_ref[t]
        return jnp.where(mask, 2.0, bto), jnp.where(mask, t, bti)

    bto, bti = jax.lax.fori_loop(0, O, _force_step, (bto, bti), unroll=25)

    def _gather_step(t, car):
        mx1, my1, mx2, my2, mlb = car
        sel = bti == t
        mx1 = jnp.where(sel, tgt_ref[0, 0, t], mx1)
        my1 = jnp.where(sel, tgt_ref[0, 1, t], my1)
        mx2 = jnp.where(sel, tgt_ref[0, 2, t], mx2)
        my2 = jnp.where(sel, tgt_ref[0, 3, t], my2)
        mlb = jnp.where(sel, tgt_ref[0, 4, t], mlb)
        return mx1, my1, mx2, my2, mlb

    z = jnp.zeros((_R, _L), jnp.float32)
    mx1, my1, mx2, my2, mlb = jax.lax.fori_loop(
        0, O, _gather_step, (z, z, z, z, z), unroll=25)

    pos = bto >= _THR
    posf = pos.astype(jnp.float32)
    conf_t = jnp.where(pos, mlb.astype(jnp.int32), 0)

    g0 = ((mx1 + mx2) * 0.5 - pcx) / (_V0 * pw)
    g1 = ((my1 + my2) * 0.5 - pcy) / (_V0 * ph)
    g2 = jnp.log((mx2 - mx1) / pw) / _V1
    g3 = jnp.log((my2 - my1) / ph) / _V1

    def _sl1(d):
        ad = jnp.abs(d)
        return jnp.where(ad < 1.0, 0.5 * d * d, ad - 0.5)

    sl1 = (_sl1(loc_ref[0, 0] - g0) + _sl1(loc_ref[0, 1] - g1) +
           _sl1(loc_ref[0, 2] - g2) + _sl1(loc_ref[0, 3] - g3))
    ll_ref[0] = jnp.sum(sl1 * posf, keepdims=True)

    _CH = 1024
    _SUB = _CH // _R
    cc_io = jax.lax.broadcasted_iota(jnp.int32, (_CH, C), 1)

    def _ce_step(i, ce_mat):
        ch = conf_ref[0, pl.ds(i * _CH, _CH), :]
        rmax = jnp.max(ch, axis=1, keepdims=True)
        lse = jnp.log(jnp.sum(jnp.exp(ch - rmax), axis=1,
                              keepdims=True)) + rmax
        cols = [jnp.sum(jnp.where(c_io == i * _SUB + j, conf_t, 0),
                        axis=1, keepdims=True) for j in range(_SUB)]
        ct_chunk = jnp.concatenate(cols, axis=0)
        gath = jnp.sum(jnp.where(cc_io == ct_chunk, ch, 0.0),
                       axis=1, keepdims=True)
        ce_col = lse - gath
        for j in range(_SUB):
            ce_mat = jnp.where(c_io == i * _SUB + j,
                               ce_col[j * _R:(j + 1) * _R, :], ce_mat)
        return ce_mat

    ce_mat = jax.lax.fori_loop(0, _R * _L // _CH, _ce_step, z, unroll=2)

    ign = ign_ref[0]
    proxy = jnp.where(pos | (ign > 0), 0.0, ce_mat)

    num_pos = jnp.sum(pos.astype(jnp.int32))
    max_neg = jnp.sum((proxy > 0.0).astype(jnp.int32))
    num_neg = jnp.minimum(_NPR * num_pos, max_neg)

    bits = jax.lax.bitcast_convert_type(proxy, jnp.int32)

    def _val_step(i, cur):
        cand = cur | jnp.left_shift(jnp.int32(1), 30 - i)
        cnt = jnp.sum((bits >= cand).astype(jnp.int32))
        return jnp.where(cnt >= num_neg, cand, cur)

    tbits = jax.lax.fori_loop(0, 31, _val_step, jnp.int32(0), unroll=31)
    cnt_gt = jnp.sum((bits > tbits).astype(jnp.int32))
    need = num_neg - cnt_gt
    tie = bits == tbits

    def _idx_step(i, cur):
        cand = cur | jnp.left_shift(jnp.int32(1), 13 - i)
        cnt = jnp.sum((tie & (lin < cand)).astype(jnp.int32))
        return jnp.where(cnt < need, cand, cur)

    tidx = jax.lax.fori_loop(0, 14, _idx_step, jnp.int32(0), unroll=14)
    neg = (bits > tbits) | (tie & (lin <= tidx) & (need > 0))
    neg = neg & (num_neg > 0)

    lc_ref[0] = jnp.sum(jnp.where(pos | neg, ce_mat, 0.0), keepdims=True)
    np_ref[0] = jnp.sum(posf, keepdims=True)


def kernel(loc_pred, conf_pred, refined_anchors, ignore_flags_refined_anchor,
           targets):
    B, A, _ = loc_pred.shape
    O = targets.shape[1]
    locT = jnp.transpose(loc_pred, (0, 2, 1)).reshape(B, 4, _R, _L).swapaxes(2, 3)
    ancT = jnp.transpose(refined_anchors, (0, 2, 1)).reshape(B, 4, _R, _L).swapaxes(2, 3)
    ignR = ignore_flags_refined_anchor.reshape(B, _R, _L).swapaxes(1, 2)
    tgtT = jnp.transpose(targets, (0, 2, 1))

    out = pl.pallas_call(
        _loss_kernel,
        grid=(B,),
        in_specs=[
            pl.BlockSpec((1, 4, _R, _L), lambda b: (b, 0, 0, 0)),
            pl.BlockSpec((1, A, _NCLS), lambda b: (b, 0, 0)),
            pl.BlockSpec((1, 4, _R, _L), lambda b: (b, 0, 0, 0)),
            pl.BlockSpec((1, _R, _L), lambda b: (b, 0, 0)),
            pl.BlockSpec((1, 5, O), lambda b: (b, 0, 0),
                         memory_space=pltpu.SMEM),
        ],
        out_specs=[
            pl.BlockSpec((1, 1, 1), lambda b: (b, 0, 0)),
            pl.BlockSpec((1, 1, 1), lambda b: (b, 0, 0)),
            pl.BlockSpec((1, 1, 1), lambda b: (b, 0, 0)),
        ],
        out_shape=[
            jax.ShapeDtypeStruct((B, 1, 1), jnp.float32),
            jax.ShapeDtypeStruct((B, 1, 1), jnp.float32),
            jax.ShapeDtypeStruct((B, 1, 1), jnp.float32),
        ],
        scratch_shapes=[pltpu.SMEM((O,), jnp.int32)],
        compiler_params=pltpu.CompilerParams(
            dimension_semantics=("parallel",),
        ),
    )(locT, conf_pred, ancT, ignR, tgtT)

    ll, lc, npos = out
    total = jnp.sum(npos)
    return (jnp.sum(ll) / total, jnp.sum(lc) / total)


# speedup vs baseline: 8.4360x; 1.0171x over previous
import jax
import jax.numpy as jnp
from jax.experimental import pallas as pl
from jax.experimental.pallas import tpu as pltpu

_NCLS = 81
_THR = 0.5
_NPR = 3
_V0, _V1 = 0.1, 0.2
_R = 128
_L = 128


def _loss_kernel(loc_ref, conf_ref, anc_ref, ign_ref, tgt_ref,
                 ll_ref, lc_ref, np_ref, bpis_ref):
    A = _R * _L
    O = tgt_ref.shape[2]
    C = conf_ref.shape[2]

    pcx = anc_ref[0, 0]
    pcy = anc_ref[0, 1]
    pw = anc_ref[0, 2]
    ph = anc_ref[0, 3]
    px1 = pcx - pw * 0.5
    py1 = pcy - ph * 0.5
    px2 = pcx + pw * 0.5
    py2 = pcy + ph * 0.5
    area_p = (px2 - px1) * (py2 - py1)

    r_io = jax.lax.broadcasted_iota(jnp.int32, (_R, _L), 0)
    c_io = jax.lax.broadcasted_iota(jnp.int32, (_R, _L), 1)
    lin = c_io * _R + r_io

    def _match_step(t, car):
        bto, bti = car
        x1 = tgt_ref[0, 0, t]
        y1 = tgt_ref[0, 1, t]
        x2 = tgt_ref[0, 2, t]
        y2 = tgt_ref[0, 3, t]
        iw = jnp.maximum(jnp.minimum(x2, px2) - jnp.maximum(x1, px1), 0.0)
        ih = jnp.maximum(jnp.minimum(y2, py2) - jnp.maximum(y1, py1), 0.0)
        inter = iw * ih
        area_t = (x2 - x1) * (y2 - y1)
        iou = inter / (area_t + area_p - inter)
        upd = iou > bto
        bti = jnp.where(upd, t, bti)
        bto = jnp.where(upd, iou, bto)
        bpo = jnp.max(iou)
        bpis_ref[t] = jnp.min(jnp.where(iou == bpo, lin, A))
        return bto, bti

    bto, bti = jax.lax.fori_loop(
        0, O, _match_step,
        (jnp.full((_R, _L), -1.0, jnp.float32),
         jnp.zeros((_R, _L), jnp.int32)), unroll=50)

    def _force_step(t, car):
        bto, bti = car
        mask = lin == bpis_ref[t]
        return jnp.where(mask, 2.0, bto), jnp.where(mask, t, bti)

    bto, bti = jax.lax.fori_loop(0, O, _force_step, (bto, bti), unroll=50)

    def _gather_step(t, car):
        mx1, my1, mx2, my2, mlb = car
        sel = bti == t
        mx1 = jnp.where(sel, tgt_ref[0, 0, t], mx1)
        my1 = jnp.where(sel, tgt_ref[0, 1, t], my1)
        mx2 = jnp.where(sel, tgt_ref[0, 2, t], mx2)
        my2 = jnp.where(sel, tgt_ref[0, 3, t], my2)
        mlb = jnp.where(sel, tgt_ref[0, 4, t], mlb)
        return mx1, my1, mx2, my2, mlb

    z = jnp.zeros((_R, _L), jnp.float32)
    mx1, my1, mx2, my2, mlb = jax.lax.fori_loop(
        0, O, _gather_step, (z, z, z, z, z), unroll=50)

    pos = bto >= _THR
    posf = pos.astype(jnp.float32)
    conf_t = jnp.where(pos, mlb.astype(jnp.int32), 0)

    g0 = ((mx1 + mx2) * 0.5 - pcx) / (_V0 * pw)
    g1 = ((my1 + my2) * 0.5 - pcy) / (_V0 * ph)
    g2 = jnp.log((mx2 - mx1) / pw) / _V1
    g3 = jnp.log((my2 - my1) / ph) / _V1

    def _sl1(d):
        ad = jnp.abs(d)
        return jnp.where(ad < 1.0, 0.5 * d * d, ad - 0.5)

    sl1 = (_sl1(loc_ref[0, 0] - g0) + _sl1(loc_ref[0, 1] - g1) +
           _sl1(loc_ref[0, 2] - g2) + _sl1(loc_ref[0, 3] - g3))
    ll_ref[0] = jnp.sum(sl1 * posf, keepdims=True)

    _CH = 1024
    _SUB = _CH // _R
    cc_io = jax.lax.broadcasted_iota(jnp.int32, (_CH, C), 1)

    def _ce_step(i, ce_mat):
        ch = conf_ref[0, pl.ds(i * _CH, _CH), :]
        rmax = jnp.max(ch, axis=1, keepdims=True)
        lse = jnp.log(jnp.sum(jnp.exp(ch - rmax), axis=1,
                              keepdims=True)) + rmax
        cols = [jnp.sum(jnp.where(c_io == i * _SUB + j, conf_t, 0),
                        axis=1, keepdims=True) for j in range(_SUB)]
        ct_chunk = jnp.concatenate(cols, axis=0)
        gath = jnp.sum(jnp.where(cc_io == ct_chunk, ch, 0.0),
                       axis=1, keepdims=True)
        ce_col = lse - gath
        for j in range(_SUB):
            ce_mat = jnp.where(c_io == i * _SUB + j,
                               ce_col[j * _R:(j + 1) * _R, :], ce_mat)
        return ce_mat

    ce_mat = jax.lax.fori_loop(0, _R * _L // _CH, _ce_step, z, unroll=2)

    ign = ign_ref[0]
    proxy = jnp.where(pos | (ign > 0), 0.0, ce_mat)

    num_pos = jnp.sum(pos.astype(jnp.int32))
    max_neg = jnp.sum((proxy > 0.0).astype(jnp.int32))
    num_neg = jnp.minimum(_NPR * num_pos, max_neg)

    bits = jax.lax.bitcast_convert_type(proxy, jnp.int32)

    def _val_step(i, cur):
        cand = cur | jnp.left_shift(jnp.int32(1), 30 - i)
        cnt = jnp.sum((bits >= cand).astype(jnp.int32))
        return jnp.where(cnt >= num_neg, cand, cur)

    tbits = jax.lax.fori_loop(0, 31, _val_step, jnp.int32(0), unroll=31)
    cnt_gt = jnp.sum((bits > tbits).astype(jnp.int32))
    need = num_neg - cnt_gt
    tie = bits == tbits

    def _idx_step(i, cur):
        cand = cur | jnp.left_shift(jnp.int32(1), 13 - i)
        cnt = jnp.sum((tie & (lin < cand)).astype(jnp.int32))
        return jnp.where(cnt < need, cand, cur)

    tidx = jax.lax.fori_loop(0, 14, _idx_step, jnp.int32(0), unroll=14)
    neg = (bits > tbits) | (tie & (lin <= tidx) & (need > 0))
    neg = neg & (num_neg > 0)

    lc_ref[0] = jnp.sum(jnp.where(pos | neg, ce_mat, 0.0), keepdims=True)
    np_ref[0] = jnp.sum(posf, keepdims=True)


def kernel(loc_pred, conf_pred, refined_anchors, ignore_flags_refined_anchor,
           targets):
    B, A, _ = loc_pred.shape
    O = targets.shape[1]
    locT = jnp.transpose(loc_pred, (0, 2, 1)).reshape(B, 4, _R, _L).swapaxes(2, 3)
    ancT = jnp.transpose(refined_anchors, (0, 2, 1)).reshape(B, 4, _R, _L).swapaxes(2, 3)
    ignR = ignore_flags_refined_anchor.reshape(B, _R, _L).swapaxes(1, 2)
    tgtT = jnp.transpose(targets, (0, 2, 1))

    out = pl.pallas_call(
        _loss_kernel,
        grid=(B,),
        in_specs=[
            pl.BlockSpec((1, 4, _R, _L), lambda b: (b, 0, 0, 0)),
            pl.BlockSpec((1, A, _NCLS), lambda b: (b, 0, 0)),
            pl.BlockSpec((1, 4, _R, _L), lambda b: (b, 0, 0, 0)),
            pl.BlockSpec((1, _R, _L), lambda b: (b, 0, 0)),
            pl.BlockSpec((1, 5, O), lambda b: (b, 0, 0),
                         memory_space=pltpu.SMEM),
        ],
        out_specs=[
            pl.BlockSpec((1, 1, 1), lambda b: (b, 0, 0)),
            pl.BlockSpec((1, 1, 1), lambda b: (b, 0, 0)),
            pl.BlockSpec((1, 1, 1), lambda b: (b, 0, 0)),
        ],
        out_shape=[
            jax.ShapeDtypeStruct((B, 1, 1), jnp.float32),
            jax.ShapeDtypeStruct((B, 1, 1), jnp.float32),
            jax.ShapeDtypeStruct((B, 1, 1), jnp.float32),
        ],
        scratch_shapes=[pltpu.SMEM((O,), jnp.int32)],
        compiler_params=pltpu.CompilerParams(
            dimension_semantics=("parallel",),
        ),
    )(locT, conf_pred, ancT, ignR, tgtT)

    ll, lc, npos = out
    total = jnp.sum(npos)
    return (jnp.sum(ll) / total, jnp.sum(lc) / total)
